# Initial kernel scaffold; baseline (speedup 1.0000x reference)
import jax
import jax.numpy as jnp
from jax.experimental import pallas as pl


def kernel(edge_index, edge_label, nft, rft, h, t, importances, trans_gnn_W, trans_gnn_b, pe_W, pe_b, attn1_W, attn2, trans_W, trans_b, lamda):
    raise NotImplementedError("write your pallas kernel here")


# speedup vs baseline: 3.5255x; 3.5255x over previous
import functools
import numpy as np
import jax
import jax.numpy as jnp
from jax import lax
from jax.experimental import pallas as pl
from jax.experimental.pallas import tpu as pltpu

L = 12
NGNN = 3
H = 8
GD = 128
HD = 768
PRE = 10
N = 2000
E = 16000
R = 64
NPAD = 2048


def _k0_body(nft_ref, rft_ref, W_ref, b_ref, x_ref, rW_ref):
    W = W_ref[0]
    b = b_ref[0]
    x_ref[0] = jnp.dot(nft_ref[0], W, preferred_element_type=jnp.float32) + b
    rW_ref[0] = jnp.dot(rft_ref[0], W, preferred_element_type=jnp.float32) + b


def _k0(nft, rft, W, b):
    return pl.pallas_call(
        _k0_body,
        grid=(L,),
        in_specs=[
            pl.BlockSpec((1, N, HD), lambda l: (l, 0, 0)),
            pl.BlockSpec((1, R, HD), lambda l: (l, 0, 0)),
            pl.BlockSpec((1, HD, GD), lambda l: (l, 0, 0)),
            pl.BlockSpec((1, 1, GD), lambda l: (l, 0, 0)),
        ],
        out_specs=[
            pl.BlockSpec((1, N, GD), lambda l: (l, 0, 0)),
            pl.BlockSpec((1, R, GD), lambda l: (l, 0, 0)),
        ],
        out_shape=[
            jax.ShapeDtypeStruct((L, N, GD), jnp.float32),
            jax.ShapeDtypeStruct((L, R, GD), jnp.float32),
        ],
    )(nft, rft, W, b.reshape(L, 1, GD))


def _k1_body(x_ref, rW_ref, pws_ref, pwr_ref, pwd_ref, a1w_ref, a2bc_ref,
             peb_ref, seg_ref, xt_ref, xb_ref, relp_ref, asrc_ref, adst_ref,
             arel_ref):
    x = x_ref[0]
    rW = rW_ref[0]
    a2bc = a2bc_ref[0]
    seg = seg_ref[...]
    xt = jnp.dot(x, pws_ref[0], preferred_element_type=jnp.float32)
    xb = jnp.dot(x, pwd_ref[0], preferred_element_type=jnp.float32)
    relp = jnp.dot(rW, pwr_ref[0], preferred_element_type=jnp.float32) + peb_ref[0]
    xt_ref[0] = xt
    xb_ref[0] = xb
    relp_ref[0] = relp
    asrc_ref[0] = (jnp.dot(x, a1w_ref[0], preferred_element_type=jnp.float32)
                   + jnp.dot(xt * a2bc, seg, preferred_element_type=jnp.float32))
    adst_ref[0] = jnp.dot(xb * a2bc, seg, preferred_element_type=jnp.float32)
    arel_ref[0] = jnp.dot(relp * a2bc, seg, preferred_element_type=jnp.float32)


def _k1(x, rW, pws, pwr, pwd, a1w16, a2bc, peb, seg16):
    blk = lambda *s: pl.BlockSpec((1,) + s, lambda l: (l,) + (0,) * len(s))
    return pl.pallas_call(
        _k1_body,
        grid=(L,),
        in_specs=[
            blk(N, GD), blk(R, GD), blk(GD, GD), blk(GD, GD), blk(GD, GD),
            blk(GD, 16), blk(1, GD), blk(1, GD),
            pl.BlockSpec((GD, 16), lambda l: (0, 0)),
        ],
        out_specs=[
            blk(N, GD), blk(N, GD), blk(R, GD), blk(N, 16), blk(N, 16),
            blk(R, 16),
        ],
        out_shape=[
            jax.ShapeDtypeStruct((L, N, GD), jnp.float32),
            jax.ShapeDtypeStruct((L, N, GD), jnp.float32),
            jax.ShapeDtypeStruct((L, R, GD), jnp.float32),
            jax.ShapeDtypeStruct((L, N, 16), jnp.float32),
            jax.ShapeDtypeStruct((L, N, 16), jnp.float32),
            jax.ShapeDtypeStruct((L, R, 16), jnp.float32),
        ],
    )(x, rW, pws, pwr, pwd, a1w16, a2bc.reshape(L, 1, GD),
      peb.reshape(L, 1, GD), seg16)


def _k1_wrap(x, rW, pws, pwr, pwd, a1w16, a2bc, peb, seg16):
    xt, xb, relp, asrc, adst, arel = _k1(x, rW, pws, pwr, pwd, a1w16, a2bc,
                                         peb, seg16)
    return xt, xb, relp, asrc, adst, arel


def _k3_body(impp_ref, x3_ref, tW_ref, tb_ref, out_ref):
    iv = impp_ref[...]
    lin = lax.broadcasted_iota(jnp.int32, (NPAD, 1), 0)
    big = jnp.int32(NPAD + 10)
    masks = []
    for k in range(PRE):
        m = jnp.max(iv)
        idxk = jnp.min(jnp.where(iv == m, lin, big))
        fmask = (lin == idxk)
        masks.append(fmask)
        iv = jnp.where(fmask, -jnp.inf, iv)
    for l in range(L):
        x3 = x3_ref[l]
        rows = []
        for k in range(PRE):
            sel = jnp.where(masks[k], x3, 0.0)
            rows.append(jnp.sum(sel, axis=0, keepdims=True))
        pref = jnp.concatenate(rows + [rows[-1]] * (16 - PRE), axis=0)
        pk = jnp.dot(jnp.tanh(pref), tW_ref[l],
                     preferred_element_type=jnp.float32) + tb_ref[l]
        out_ref[l] = pk


def _k3(impp, x3p, tW, tb):
    OUT2 = tW.shape[-1]
    return pl.pallas_call(
        _k3_body,
        out_shape=jax.ShapeDtypeStruct((L, 16, OUT2), jnp.float32),
    )(impp, x3p, tW, tb.reshape(L, 1, OUT2))


def _edges_jnp(xt, xb, relp, asrc, adst, arel, src, dst, lab, x):
    FH = GD // H
    a = asrc[:, src, :H] + arel[:, lab, :H] + adst[:, dst, :H]
    a = jnp.where(a > 0, a, 0.01 * a)
    m = jax.vmap(lambda aa: jax.ops.segment_max(aa, dst, num_segments=N))(a)
    m = jnp.where(jnp.isfinite(m), m, 0.0)
    e = jnp.exp(a - m[:, dst])
    ssum = jax.vmap(lambda ee: jax.ops.segment_sum(ee, dst, num_segments=N))(e)
    ep = xt[:, src] + relp[:, lab]
    wep = ep * jnp.repeat(e, FH, axis=-1)
    num = jax.vmap(lambda w: jax.ops.segment_sum(w, dst, num_segments=N))(wep)
    ssum_b = jnp.repeat(ssum, FH, axis=-1)
    ft = jnp.where(ssum_b > 0, num / jnp.where(ssum_b > 0, ssum_b, 1.0) + xb,
                   0.0)
    return jnp.maximum(ft + x, 0.0)


def kernel(edge_index, edge_label, nft, rft, h, t, importances, trans_gnn_W,
           trans_gnn_b, pe_W, pe_b, attn1_W, attn2, trans_W, trans_b, lamda):
    src = edge_index[0]
    dst = edge_index[1]
    lab = edge_label

    seg16 = jnp.asarray(np.kron(np.eye(H, dtype=np.float32),
                                np.ones((GD // H, 1), np.float32)))
    seg16 = jnp.pad(seg16, ((0, 0), (0, 16 - H)))
    a2bc = attn2.reshape(L, NGNN, GD)
    a1w16 = jnp.pad(attn1_W, ((0, 0), (0, 0), (0, 0), (0, 16 - H)))
    pws = pe_W[:, :, :GD]
    pwr = pe_W[:, :, GD:2 * GD]
    pwd = pe_W[:, :, 2 * GD:]

    x, rW = _k0(nft, rft, trans_gnn_W, trans_gnn_b)

    for j in range(NGNN):
        xt, xb, relp, asrc, adst, arel = _k1_wrap(
            x, rW, pws[:, j], pwr[:, j], pwd[:, j], a1w16[:, j], a2bc[:, j],
            pe_b[:, j], seg16)
        x = _edges_jnp(xt, xb, relp, asrc, adst, arel, src, dst, lab, x)

    impp = jnp.pad(importances, (0, NPAD - N),
                   constant_values=-jnp.inf).reshape(NPAD, 1)
    x3p = jnp.pad(x, ((0, 0), (0, NPAD - N), (0, 0)))
    pk_all = _k3(impp, x3p, trans_W, trans_b)[:, :PRE]

    OUT2 = trans_W.shape[-1]
    OUT = OUT2 // 2
    KVH = 4
    B = h.shape[0]
    out = []
    for i in range(L):
        pk = jnp.broadcast_to(pk_all[i][None], (B, PRE, OUT2))
        pk = pk.reshape(B, PRE, 2, KVH, OUT // KVH).transpose(2, 0, 3, 1, 4)
        out.append(pk)
    return tuple(out)


# speedup vs baseline: 13.6030x; 3.8584x over previous
import functools
import numpy as np
import jax
import jax.numpy as jnp
from jax import lax
from jax.experimental import pallas as pl
from jax.experimental.pallas import tpu as pltpu
from jax.experimental.pallas import tpu_sc as plsc

L = 12
NGNN = 3
H = 8
GD = 128
HD = 768
PRE = 10
N = 2000
E = 16000
R = 64
NPAD = 2048
N2 = 2048
GW = 256


def _k0_body(nft_ref, rft_ref, W_ref, b_ref, x_ref, rW_ref):
    W = W_ref[0]
    b = b_ref[0]
    x_ref[0] = jnp.dot(nft_ref[0], W, preferred_element_type=jnp.float32) + b
    rW_ref[0] = jnp.dot(rft_ref[0], W, preferred_element_type=jnp.float32) + b


def _k0(nft, rft, W, b):
    return pl.pallas_call(
        _k0_body,
        grid=(L,),
        in_specs=[
            pl.BlockSpec((1, N2, HD), lambda l: (l, 0, 0)),
            pl.BlockSpec((1, R, HD), lambda l: (l, 0, 0)),
            pl.BlockSpec((1, HD, GD), lambda l: (l, 0, 0)),
            pl.BlockSpec((1, 1, GD), lambda l: (l, 0, 0)),
        ],
        out_specs=[
            pl.BlockSpec((1, N2, GD), lambda l: (l, 0, 0)),
            pl.BlockSpec((1, R, GD), lambda l: (l, 0, 0)),
        ],
        out_shape=[
            jax.ShapeDtypeStruct((L, N2, GD), jnp.float32),
            jax.ShapeDtypeStruct((L, R, GD), jnp.float32),
        ],
    )(nft, rft, W, b.reshape(L, 1, GD))


def _k1_body(x_ref, rW_ref, pws_ref, pwr_ref, pwd_ref, a1w_ref, a2bc_ref,
             peb_ref, seg_ref, xta_ref, relpa_ref, xb_ref, adst_ref):
    x = x_ref[0]
    rW = rW_ref[0]
    a2bc = a2bc_ref[0]
    seg = seg_ref[...]
    xt = jnp.dot(x, pws_ref[0], preferred_element_type=jnp.float32)
    xb = jnp.dot(x, pwd_ref[0], preferred_element_type=jnp.float32)
    relp = jnp.dot(rW, pwr_ref[0], preferred_element_type=jnp.float32) + peb_ref[0]
    asrc = (jnp.dot(x, a1w_ref[0], preferred_element_type=jnp.float32)
            + jnp.dot(xt * a2bc, seg, preferred_element_type=jnp.float32))
    arel = jnp.dot(relp * a2bc, seg, preferred_element_type=jnp.float32)
    adst = jnp.dot(xb * a2bc, seg, preferred_element_type=jnp.float32)
    zn = jnp.zeros((N2, GD - 16), jnp.float32)
    zr = jnp.zeros((R, GD - 16), jnp.float32)
    xta_ref[0] = jnp.concatenate([xt, asrc, zn], axis=1)
    relpa_ref[0] = jnp.concatenate([relp, arel, zr], axis=1)
    xb_ref[0] = xb
    adst_ref[0] = adst


def _k1(x, rW, pws, pwr, pwd, a1w16, a2bc, peb, seg16):
    blk = lambda *s: pl.BlockSpec((1,) + s, lambda l: (l,) + (0,) * len(s))
    return pl.pallas_call(
        _k1_body,
        grid=(L,),
        in_specs=[
            blk(N2, GD), blk(R, GD), blk(GD, GD), blk(GD, GD), blk(GD, GD),
            blk(GD, 16), blk(1, GD), blk(1, GD),
            pl.BlockSpec((GD, 16), lambda l: (0, 0)),
        ],
        out_specs=[
            blk(N2, GW), blk(R, GW), blk(N2, GD), blk(N2, 16),
        ],
        out_shape=[
            jax.ShapeDtypeStruct((L, N2, GW), jnp.float32),
            jax.ShapeDtypeStruct((L, R, GW), jnp.float32),
            jax.ShapeDtypeStruct((L, N2, GD), jnp.float32),
            jax.ShapeDtypeStruct((L, N2, 16), jnp.float32),
        ],
    )(x, rW, pws, pwr, pwd, a1w16, a2bc.reshape(L, 1, GD),
      peb.reshape(L, 1, GD), seg16)


def _k3_body(impp_ref, x3_ref, tW_ref, tb_ref, out_ref):
    iv = impp_ref[...]
    lin = lax.broadcasted_iota(jnp.int32, (NPAD, 1), 0)
    big = jnp.int32(NPAD + 10)
    masks = []
    for k in range(PRE):
        m = jnp.max(iv)
        idxk = jnp.min(jnp.where(iv == m, lin, big))
        fmask = (lin == idxk)
        masks.append(fmask)
        iv = jnp.where(fmask, -jnp.inf, iv)
    for l in range(L):
        x3 = x3_ref[l]
        rows = []
        for k in range(PRE):
            sel = jnp.where(masks[k], x3, 0.0)
            rows.append(jnp.sum(sel, axis=0, keepdims=True))
        pref = jnp.concatenate(rows + [rows[-1]] * (16 - PRE), axis=0)
        pk = jnp.dot(jnp.tanh(pref), tW_ref[l],
                     preferred_element_type=jnp.float32) + tb_ref[l]
        out_ref[l] = pk


def _k3(impp, x3p, tW, tb):
    OUT2 = tW.shape[-1]
    return pl.pallas_call(
        _k3_body,
        out_shape=jax.ShapeDtypeStruct((L, 16, OUT2), jnp.float32),
    )(impp, x3p, tW, tb.reshape(L, 1, OUT2))


BE = 64
C = 32
NCK = N2 // C
NT = L * C
TPT = NT // 32


def _k2_body(xta_h, relpa_h, adst_h, xb_h, x_h, srcp_h, labp_h, dstp_h,
             bounds_h, zer128_h, mneg_h, zer16_h,
             xout_h,
             brow_v, src_v, lab_v, dstl_v, gixt_v, girel_v,
             xtag_v, relag_v,
             xloc_v, xbloc_v, num_v, adl_v, m_v, ssum_v):
    ncores = 2
    wid = lax.axis_index("s") * ncores + lax.axis_index("c")

    def _task(tt, _carry):
        tsk = tt * 32 + wid
        pltpu.sync_copy(bounds_h.at[pl.ds(pl.multiple_of(tsk * 16, 8), 16)],
                        brow_v)
        br = brow_v[...]
        e0a = br[0]
        e1 = br[1]
        nb = br[2]
        cbn = br[3]
        rb = br[4]
        e0 = br[5]
        cb = pl.multiple_of(nb + cbn, 8)

        pltpu.sync_copy(zer128_h, num_v)
        pltpu.sync_copy(mneg_h, m_v)
        pltpu.sync_copy(zer16_h, ssum_v)
        pltpu.sync_copy(x_h.at[pl.ds(cb, NCK)], xloc_v)
        pltpu.sync_copy(xb_h.at[pl.ds(cb, NCK)], xbloc_v)
        pltpu.sync_copy(adst_h.at[pl.ds(cb, NCK)], adl_v)

        nblk = (e1 - e0a + BE - 1) // BE
        nb_v = jnp.full((16,), nb, jnp.int32)
        rb_v = jnp.full((16,), rb, jnp.int32)
        cbn_v = jnp.full((16,), cbn, jnp.int32)

        def stage_idx(b):
            eb0 = pl.multiple_of(e0a + b * BE, 8)
            pltpu.sync_copy(srcp_h.at[pl.ds(eb0, BE)], src_v.at[pl.ds(0, BE)])
            pltpu.sync_copy(labp_h.at[pl.ds(eb0, BE)], lab_v.at[pl.ds(0, BE)])
            pltpu.sync_copy(dstp_h.at[pl.ds(eb0, BE)], dstl_v.at[pl.ds(0, BE)])

            def mkidx(i, _):
                sl = pl.ds(i * 16, 16)
                gixt_v[sl] = src_v[sl] + nb_v
                girel_v[sl] = lab_v[sl] + rb_v
                dstl_v[sl] = dstl_v[sl] - cbn_v
                return 0

            lax.fori_loop(0, BE // 16, mkidx, 0)
            pltpu.sync_copy(xta_h.at[gixt_v], xtag_v)
            pltpu.sync_copy(relpa_h.at[girel_v], relag_v)
            lo = jnp.maximum(e0 - eb0, 0)
            hi = jnp.minimum(e1 - eb0, BE)
            return lo, hi

        def p1_blk(b, _):
            lo, hi = stage_idx(b)

            def p1_edge(e, _):
                dl = dstl_v[pl.ds(e, 16)][0]
                av = (xtag_v[e, pl.ds(GD, 16)] + relag_v[e, pl.ds(GD, 16)]
                      + adl_v[dl, pl.ds(0, 16)])
                av = jnp.maximum(av, 0.01 * av)
                m_v[dl, pl.ds(0, 16)] = jnp.maximum(m_v[dl, pl.ds(0, 16)], av)
                return 0

            lax.fori_loop(lo, hi, p1_edge, 0)
            return 0

        lax.fori_loop(0, nblk, p1_blk, 0)

        def p2_blk(b, _):
            lo, hi = stage_idx(b)

            def p2_edge(e, _):
                dl = dstl_v[pl.ds(e, 16)][0]
                av = (xtag_v[e, pl.ds(GD, 16)] + relag_v[e, pl.ds(GD, 16)]
                      + adl_v[dl, pl.ds(0, 16)])
                av = jnp.maximum(av, 0.01 * av)
                ev = jnp.exp(av - m_v[dl, pl.ds(0, 16)])
                ssum_v[dl, pl.ds(0, 16)] = ssum_v[dl, pl.ds(0, 16)] + ev
                for hh in range(H):
                    cs = pl.ds(16 * hh, 16)
                    ep = xtag_v[e, cs] + relag_v[e, cs]
                    num_v[dl, cs] = num_v[dl, cs] + ep * ev[hh]
                return 0

            lax.fori_loop(lo, hi, p2_edge, 0)
            return 0

        lax.fori_loop(0, nblk, p2_blk, 0)

        def fin(n, _):
            ss = ssum_v[n, pl.ds(0, 16)]
            for hh in range(H):
                cs = pl.ds(16 * hh, 16)
                sh = ss[hh]
                ft = jnp.where(sh > 0.0,
                               num_v[n, cs] / jnp.maximum(sh, 1e-30)
                               + xbloc_v[n, cs], 0.0)
                num_v[n, cs] = jnp.maximum(ft + xloc_v[n, cs], 0.0)
            return 0

        lax.fori_loop(0, NCK, fin, 0)
        pltpu.sync_copy(num_v, xout_h.at[pl.ds(cb, NCK)])
        return 0

    lax.fori_loop(0, TPT, _task, 0)


def _k2(xta, relpa, adst, xb, x, srcp, labp, dstp, bounds, zer128, mneg,
        zer16):
    mesh = plsc.VectorSubcoreMesh(core_axis_name="c", subcore_axis_name="s")
    f = pl.kernel(
        _k2_body,
        mesh=mesh,
        out_type=jax.ShapeDtypeStruct((L * N2, GD), jnp.float32),
        scratch_types=[
            pltpu.VMEM((16,), jnp.int32),
            pltpu.VMEM((BE + 16,), jnp.int32),
            pltpu.VMEM((BE + 16,), jnp.int32),
            pltpu.VMEM((BE + 16,), jnp.int32),
            pltpu.VMEM((BE,), jnp.int32),
            pltpu.VMEM((BE,), jnp.int32),
            pltpu.VMEM((BE, GW), jnp.float32),
            pltpu.VMEM((BE, GW), jnp.float32),
            pltpu.VMEM((NCK, GD), jnp.float32),
            pltpu.VMEM((NCK, GD), jnp.float32),
            pltpu.VMEM((NCK, GD), jnp.float32),
            pltpu.VMEM((NCK, 16), jnp.float32),
            pltpu.VMEM((NCK, 16), jnp.float32),
            pltpu.VMEM((NCK, 16), jnp.float32),
        ],
    )
    return f(xta, relpa, adst, xb, x, srcp, labp, dstp, bounds, zer128, mneg,
             zer16)


def kernel(edge_index, edge_label, nft, rft, h, t, importances, trans_gnn_W,
           trans_gnn_b, pe_W, pe_b, attn1_W, attn2, trans_W, trans_b, lamda):
    src = edge_index[0]
    dst = edge_index[1]
    lab = edge_label

    seg16 = jnp.asarray(np.kron(np.eye(H, dtype=np.float32),
                                np.ones((GD // H, 1), np.float32)))
    seg16 = jnp.pad(seg16, ((0, 0), (0, 16 - H)))
    a2bc = attn2.reshape(L, NGNN, GD)
    a1w16 = jnp.pad(attn1_W, ((0, 0), (0, 0), (0, 0), (0, 16 - H)))
    pws = pe_W[:, :, :GD]
    pwr = pe_W[:, :, GD:2 * GD]
    pwd = pe_W[:, :, 2 * GD:]

    order = jnp.argsort(dst)
    src_s = src[order]
    dst_s = dst[order]
    lab_s = lab[order]
    srcp = jnp.pad(src_s, (0, BE + 16)).astype(jnp.int32)
    dstp = jnp.pad(dst_s, (0, BE + 16)).astype(jnp.int32)
    labp = jnp.pad(lab_s, (0, BE + 16)).astype(jnp.int32)
    bnd = jnp.searchsorted(dst_s, jnp.arange(C + 1) * NCK).astype(jnp.int32)
    e0 = bnd[:-1]
    e1 = bnd[1:]
    e0a = (e0 // 8) * 8
    ll = jnp.repeat(jnp.arange(L, dtype=jnp.int32), C)
    cc = jnp.tile(jnp.arange(C, dtype=jnp.int32), L)
    rows = jnp.stack([e0a[cc], e1[cc], ll * N2, cc * NCK, ll * R,
                      e0[cc]] + [jnp.zeros(NT, jnp.int32)] * 10, axis=1)
    bounds = rows.reshape(NT * 16)
    zer128 = jnp.zeros((NCK, GD), jnp.float32)
    mneg = jnp.full((NCK, 16), -1e30, jnp.float32)
    zer16 = jnp.zeros((NCK, 16), jnp.float32)

    nftp = jnp.pad(nft, ((0, 0), (0, N2 - N), (0, 0)))
    x, rW = _k0(nftp, rft, trans_gnn_W, trans_gnn_b)

    for j in range(NGNN):
        xta, relpa, xb, adst = _k1(
            x, rW, pws[:, j], pwr[:, j], pwd[:, j], a1w16[:, j], a2bc[:, j],
            pe_b[:, j], seg16)
        xn = _k2(xta.reshape(L * N2, GW), relpa.reshape(L * R, GW),
                 adst.reshape(L * N2, 16), xb.reshape(L * N2, GD),
                 x.reshape(L * N2, GD), srcp, labp, dstp, bounds,
                 zer128, mneg, zer16)
        x = xn.reshape(L, N2, GD)

    impp = jnp.pad(importances, (0, NPAD - N),
                   constant_values=-jnp.inf).reshape(NPAD, 1)
    pk_all = _k3(impp, x, trans_W, trans_b)[:, :PRE]

    OUT2 = trans_W.shape[-1]
    OUT = OUT2 // 2
    KVH = 4
    B = h.shape[0]
    out = []
    for i in range(L):
        pk = jnp.broadcast_to(pk_all[i][None], (B, PRE, OUT2))
        pk = pk.reshape(B, PRE, 2, KVH, OUT // KVH).transpose(2, 0, 3, 1, 4)
        out.append(pk)
    return tuple(out)


# speedup vs baseline: 15.9948x; 1.1758x over previous
import functools
import numpy as np
import jax
import jax.numpy as jnp
from jax import lax
from jax.experimental import pallas as pl
from jax.experimental.pallas import tpu as pltpu
from jax.experimental.pallas import tpu_sc as plsc

L = 12
NGNN = 3
H = 8
GD = 128
HD = 768
PRE = 10
N = 2000
E = 16000
R = 64
NPAD = 2048
N2 = 2048
GW = 256


def _k0_body(nft_ref, rft_ref, W_ref, b_ref, x_ref, rW_ref):
    W = W_ref[0]
    b = b_ref[0]
    x_ref[0] = jnp.dot(nft_ref[0], W, preferred_element_type=jnp.float32) + b
    rW_ref[0] = jnp.dot(rft_ref[0], W, preferred_element_type=jnp.float32) + b


def _k0(nft, rft, W, b):
    return pl.pallas_call(
        _k0_body,
        grid=(L,),
        in_specs=[
            pl.BlockSpec((1, N2, HD), lambda l: (l, 0, 0)),
            pl.BlockSpec((1, R, HD), lambda l: (l, 0, 0)),
            pl.BlockSpec((1, HD, GD), lambda l: (l, 0, 0)),
            pl.BlockSpec((1, 1, GD), lambda l: (l, 0, 0)),
        ],
        out_specs=[
            pl.BlockSpec((1, N2, GD), lambda l: (l, 0, 0)),
            pl.BlockSpec((1, R, GD), lambda l: (l, 0, 0)),
        ],
        out_shape=[
            jax.ShapeDtypeStruct((L, N2, GD), jnp.float32),
            jax.ShapeDtypeStruct((L, R, GD), jnp.float32),
        ],
    )(nft, rft, W, b.reshape(L, 1, GD))


def _k1_body(x_ref, rW_ref, pws_ref, pwr_ref, pwd_ref, a1w_ref, a2bc_ref,
             peb_ref, seg_ref, xta_ref, relpa_ref, xb_ref, adst_ref):
    x = x_ref[0]
    rW = rW_ref[0]
    a2bc = a2bc_ref[0]
    seg = seg_ref[...]
    xt = jnp.dot(x, pws_ref[0], preferred_element_type=jnp.float32)
    xb = jnp.dot(x, pwd_ref[0], preferred_element_type=jnp.float32)
    relp = jnp.dot(rW, pwr_ref[0], preferred_element_type=jnp.float32) + peb_ref[0]
    asrc = (jnp.dot(x, a1w_ref[0], preferred_element_type=jnp.float32)
            + jnp.dot(xt * a2bc, seg, preferred_element_type=jnp.float32))
    arel = jnp.dot(relp * a2bc, seg, preferred_element_type=jnp.float32)
    adst = jnp.dot(xb * a2bc, seg, preferred_element_type=jnp.float32)
    zn = jnp.zeros((N2, GD - 16), jnp.float32)
    zr = jnp.zeros((R, GD - 16), jnp.float32)
    xta_ref[0] = jnp.concatenate([xt, asrc, zn], axis=1)
    relpa_ref[0] = jnp.concatenate([relp, arel, zr], axis=1)
    xb_ref[0] = xb
    adst_ref[0] = adst


def _k1(x, rW, pws, pwr, pwd, a1w16, a2bc, peb, seg16):
    blk = lambda *s: pl.BlockSpec((1,) + s, lambda l: (l,) + (0,) * len(s))
    return pl.pallas_call(
        _k1_body,
        grid=(L,),
        in_specs=[
            blk(N2, GD), blk(R, GD), blk(GD, GD), blk(GD, GD), blk(GD, GD),
            blk(GD, 16), blk(1, GD), blk(1, GD),
            pl.BlockSpec((GD, 16), lambda l: (0, 0)),
        ],
        out_specs=[
            blk(N2, GW), blk(R, GW), blk(N2, GD), blk(N2, 16),
        ],
        out_shape=[
            jax.ShapeDtypeStruct((L, N2, GW), jnp.float32),
            jax.ShapeDtypeStruct((L, R, GW), jnp.float32),
            jax.ShapeDtypeStruct((L, N2, GD), jnp.float32),
            jax.ShapeDtypeStruct((L, N2, 16), jnp.float32),
        ],
    )(x, rW, pws, pwr, pwd, a1w16, a2bc.reshape(L, 1, GD),
      peb.reshape(L, 1, GD), seg16)


def _k3_body(impp_ref, x3_ref, tW_ref, tb_ref, out_ref):
    iv = impp_ref[...]
    lin = lax.broadcasted_iota(jnp.int32, (NPAD, 1), 0)
    big = jnp.int32(NPAD + 10)
    masks = []
    for k in range(PRE):
        m = jnp.max(iv)
        idxk = jnp.min(jnp.where(iv == m, lin, big))
        fmask = (lin == idxk)
        masks.append(fmask)
        iv = jnp.where(fmask, -jnp.inf, iv)
    for l in range(L):
        x3 = x3_ref[l]
        rows = []
        for k in range(PRE):
            sel = jnp.where(masks[k], x3, 0.0)
            rows.append(jnp.sum(sel, axis=0, keepdims=True))
        pref = jnp.concatenate(rows + [rows[-1]] * (16 - PRE), axis=0)
        pk = jnp.dot(jnp.tanh(pref), tW_ref[l],
                     preferred_element_type=jnp.float32) + tb_ref[l]
        out_ref[l] = pk


def _k3(impp, x3p, tW, tb):
    OUT2 = tW.shape[-1]
    return pl.pallas_call(
        _k3_body,
        out_shape=jax.ShapeDtypeStruct((L, 16, OUT2), jnp.float32),
    )(impp, x3p, tW, tb.reshape(L, 1, OUT2))


BE = 96
C = 32
NCK = N2 // C
NT = L * C
TPT = NT // 32


def _k2_body(xta_h, relpa_h, adst_h, xb_h, x_h, srcp_h, labp_h, dstp_h,
             bounds_h, zer128_h, mneg_h, zer16_h,
             xout_h,
             brow_v, src_v, lab_v, dstl_v, gixt_v, girel_v,
             xtag_v, relag_v, sem1_v, sem2_v,
             xloc_v, xbloc_v, num_v, adl_v, m_v, ssum_v):
    ncores = 2
    wid = lax.axis_index("s") * ncores + lax.axis_index("c")

    def _task(tt, _carry):
        tsk = tt * 32 + wid
        pltpu.sync_copy(bounds_h.at[pl.ds(pl.multiple_of(tsk * 16, 8), 16)],
                        brow_v)
        br = brow_v[...]
        e0a = br[0]
        e1 = br[1]
        nb = br[2]
        cbn = br[3]
        rb = br[4]
        e0 = br[5]
        cb = pl.multiple_of(nb + cbn, 8)

        pltpu.sync_copy(zer128_h, num_v)
        pltpu.sync_copy(mneg_h, m_v)
        pltpu.sync_copy(zer16_h, ssum_v)
        pltpu.sync_copy(x_h.at[pl.ds(cb, NCK)], xloc_v)
        pltpu.sync_copy(xb_h.at[pl.ds(cb, NCK)], xbloc_v)
        pltpu.sync_copy(adst_h.at[pl.ds(cb, NCK)], adl_v)

        nblk = (e1 - e0a + BE - 1) // BE
        nb_v = jnp.full((16,), nb, jnp.int32)
        rb_v = jnp.full((16,), rb, jnp.int32)
        cbn_v = jnp.full((16,), cbn, jnp.int32)

        def stage_idx(b):
            eb0 = pl.multiple_of(e0a + b * BE, 8)
            pltpu.sync_copy(srcp_h.at[pl.ds(eb0, BE)], src_v.at[pl.ds(0, BE)])
            pltpu.sync_copy(labp_h.at[pl.ds(eb0, BE)], lab_v.at[pl.ds(0, BE)])
            pltpu.sync_copy(dstp_h.at[pl.ds(eb0, BE)], dstl_v.at[pl.ds(0, BE)])

            def mkidx(i, _):
                sl = pl.ds(i * 16, 16)
                gixt_v[sl] = src_v[sl] + nb_v
                girel_v[sl] = lab_v[sl] + rb_v
                dstl_v[sl] = dstl_v[sl] - cbn_v
                return 0

            lax.fori_loop(0, BE // 16, mkidx, 0)
            cp1 = pltpu.async_copy(xta_h.at[gixt_v], xtag_v, sem1_v)
            cp2 = pltpu.async_copy(relpa_h.at[girel_v], relag_v, sem2_v)
            cp1.wait()
            cp2.wait()
            lo = jnp.maximum(e0 - eb0, 0)
            hi = jnp.minimum(e1 - eb0, BE)
            return lo, hi

        def p1_blk(b, _):
            lo, hi = stage_idx(b)

            def p1_edge(e, _):
                dl = dstl_v[pl.ds(e, 16)][0]
                av = (xtag_v[e, pl.ds(GD, 16)] + relag_v[e, pl.ds(GD, 16)]
                      + adl_v[dl, pl.ds(0, 16)])
                av = jnp.maximum(av, 0.01 * av)
                m_v[dl, pl.ds(0, 16)] = jnp.maximum(m_v[dl, pl.ds(0, 16)], av)
                return 0

            lax.fori_loop(lo, hi, p1_edge, 0)
            return 0

        lax.fori_loop(0, nblk, p1_blk, 0)

        def p2_blk(b, _):
            lo, hi = stage_idx(b)

            def p2_edge(e, _):
                dl = dstl_v[pl.ds(e, 16)][0]
                av = (xtag_v[e, pl.ds(GD, 16)] + relag_v[e, pl.ds(GD, 16)]
                      + adl_v[dl, pl.ds(0, 16)])
                av = jnp.maximum(av, 0.01 * av)
                ev = jnp.exp(av - m_v[dl, pl.ds(0, 16)])
                ssum_v[dl, pl.ds(0, 16)] = ssum_v[dl, pl.ds(0, 16)] + ev
                for hh in range(H):
                    cs = pl.ds(16 * hh, 16)
                    ep = xtag_v[e, cs] + relag_v[e, cs]
                    num_v[dl, cs] = num_v[dl, cs] + ep * ev[hh]
                return 0

            lax.fori_loop(lo, hi, p2_edge, 0)
            return 0

        lax.fori_loop(0, nblk, p2_blk, 0)

        def fin(n, _):
            ss = ssum_v[n, pl.ds(0, 16)]
            for hh in range(H):
                cs = pl.ds(16 * hh, 16)
                sh = ss[hh]
                ft = jnp.where(sh > 0.0,
                               num_v[n, cs] / jnp.maximum(sh, 1e-30)
                               + xbloc_v[n, cs], 0.0)
                num_v[n, cs] = jnp.maximum(ft + xloc_v[n, cs], 0.0)
            return 0

        lax.fori_loop(0, NCK, fin, 0)
        pltpu.sync_copy(num_v, xout_h.at[pl.ds(cb, NCK)])
        return 0

    lax.fori_loop(0, TPT, _task, 0)


def _k2(xta, relpa, adst, xb, x, srcp, labp, dstp, bounds, zer128, mneg,
        zer16):
    mesh = plsc.VectorSubcoreMesh(core_axis_name="c", subcore_axis_name="s")
    f = pl.kernel(
        _k2_body,
        mesh=mesh,
        out_type=jax.ShapeDtypeStruct((L * N2, GD), jnp.float32),
        scratch_types=[
            pltpu.VMEM((16,), jnp.int32),
            pltpu.VMEM((BE + 16,), jnp.int32),
            pltpu.VMEM((BE + 16,), jnp.int32),
            pltpu.VMEM((BE + 16,), jnp.int32),
            pltpu.VMEM((BE,), jnp.int32),
            pltpu.VMEM((BE,), jnp.int32),
            pltpu.VMEM((BE, GW), jnp.float32),
            pltpu.VMEM((BE, GW), jnp.float32),
            pltpu.SemaphoreType.DMA,
            pltpu.SemaphoreType.DMA,
            pltpu.VMEM((NCK, GD), jnp.float32),
            pltpu.VMEM((NCK, GD), jnp.float32),
            pltpu.VMEM((NCK, GD), jnp.float32),
            pltpu.VMEM((NCK, 16), jnp.float32),
            pltpu.VMEM((NCK, 16), jnp.float32),
            pltpu.VMEM((NCK, 16), jnp.float32),
        ],
    )
    return f(xta, relpa, adst, xb, x, srcp, labp, dstp, bounds, zer128, mneg,
             zer16)


def kernel(edge_index, edge_label, nft, rft, h, t, importances, trans_gnn_W,
           trans_gnn_b, pe_W, pe_b, attn1_W, attn2, trans_W, trans_b, lamda):
    src = edge_index[0]
    dst = edge_index[1]
    lab = edge_label

    seg16 = jnp.asarray(np.kron(np.eye(H, dtype=np.float32),
                                np.ones((GD // H, 1), np.float32)))
    seg16 = jnp.pad(seg16, ((0, 0), (0, 16 - H)))
    a2bc = attn2.reshape(L, NGNN, GD)
    a1w16 = jnp.pad(attn1_W, ((0, 0), (0, 0), (0, 0), (0, 16 - H)))
    pws = pe_W[:, :, :GD]
    pwr = pe_W[:, :, GD:2 * GD]
    pwd = pe_W[:, :, 2 * GD:]

    order = jnp.argsort(dst)
    src_s = src[order]
    dst_s = dst[order]
    lab_s = lab[order]
    srcp = jnp.pad(src_s, (0, BE + 16)).astype(jnp.int32)
    dstp = jnp.pad(dst_s, (0, BE + 16)).astype(jnp.int32)
    labp = jnp.pad(lab_s, (0, BE + 16)).astype(jnp.int32)
    bnd = jnp.searchsorted(dst_s, jnp.arange(C + 1) * NCK).astype(jnp.int32)
    e0 = bnd[:-1]
    e1 = bnd[1:]
    e0a = (e0 // 8) * 8
    ll = jnp.repeat(jnp.arange(L, dtype=jnp.int32), C)
    cc = jnp.tile(jnp.arange(C, dtype=jnp.int32), L)
    rows = jnp.stack([e0a[cc], e1[cc], ll * N2, cc * NCK, ll * R,
                      e0[cc]] + [jnp.zeros(NT, jnp.int32)] * 10, axis=1)
    bounds = rows.reshape(NT * 16)
    zer128 = jnp.zeros((NCK, GD), jnp.float32)
    mneg = jnp.full((NCK, 16), -1e30, jnp.float32)
    zer16 = jnp.zeros((NCK, 16), jnp.float32)

    nftp = jnp.pad(nft, ((0, 0), (0, N2 - N), (0, 0)))
    x, rW = _k0(nftp, rft, trans_gnn_W, trans_gnn_b)

    for j in range(NGNN):
        xta, relpa, xb, adst = _k1(
            x, rW, pws[:, j], pwr[:, j], pwd[:, j], a1w16[:, j], a2bc[:, j],
            pe_b[:, j], seg16)
        xn = _k2(xta.reshape(L * N2, GW), relpa.reshape(L * R, GW),
                 adst.reshape(L * N2, 16), xb.reshape(L * N2, GD),
                 x.reshape(L * N2, GD), srcp, labp, dstp, bounds,
                 zer128, mneg, zer16)
        x = xn.reshape(L, N2, GD)

    impp = jnp.pad(importances, (0, NPAD - N),
                   constant_values=-jnp.inf).reshape(NPAD, 1)
    pk_all = _k3(impp, x, trans_W, trans_b)[:, :PRE]

    OUT2 = trans_W.shape[-1]
    OUT = OUT2 // 2
    KVH = 4
    B = h.shape[0]
    out = []
    for i in range(L):
        pk = jnp.broadcast_to(pk_all[i][None], (B, PRE, OUT2))
        pk = pk.reshape(B, PRE, 2, KVH, OUT // KVH).transpose(2, 0, 3, 1, 4)
        out.append(pk)
    return tuple(out)
